# baseline (speedup 1.0000x reference)
import functools

import jax
import jax.numpy as jnp
from jax import lax
from jax.experimental import pallas as pl
from jax.experimental.pallas import tpu as pltpu
from jax.experimental.pallas import tpu_sc as plsc

N = 10000
E = 320000
D = 128
G = 256

NC = 2
NS = 16
NW = NC * NS

CHUNK = 128
NCHUNK = 80
NPAIR = NCHUNK // 2
E_PAD = NC * NS * NCHUNK * CHUNK
N_PAD = 10240
ROWS_PER_TILE = N_PAD // NS
BN = 1000
GRID = N // BN


_sc_mesh = plsc.VectorSubcoreMesh(core_axis_name="c", subcore_axis_name="s")


@functools.partial(
    pl.kernel,
    out_type=jax.ShapeDtypeStruct((NC, N_PAD, D), jnp.float32),
    mesh=_sc_mesh,
    scratch_types=[
        pltpu.VMEM((2, 2, CHUNK), jnp.int32),
        pltpu.VMEM((2, 2, CHUNK), jnp.int32),
        pltpu.VMEM((CHUNK, D), jnp.float32),
        pltpu.VMEM((CHUNK, D), jnp.float32),
        pltpu.VMEM_SHARED((N_PAD, D), jnp.float32),
        pltpu.SemaphoreType.DMA,
        pltpu.SemaphoreType.DMA,
        pltpu.SemaphoreType.DMA,
        pltpu.SemaphoreType.DMA,
    ],
)
def _sc_scatter(m_hbm, idx_hbm, out_hbm,
                pbuf_a, pbuf_b, rows_a, rows_b, agg_s,
                sem_a, sem_b, sem_ia, sem_ib):
    c = lax.axis_index("c")
    s = lax.axis_index("s")
    row0 = s * ROWS_PER_TILE
    rows_a[...] = jnp.zeros((CHUNK, D), jnp.float32)
    for k in range(ROWS_PER_TILE // CHUNK):
        pltpu.sync_copy(rows_a, agg_s.at[pl.ds(row0 + k * CHUNK, CHUNK)])
    plsc.subcore_barrier()

    pltpu.sync_copy(idx_hbm.at[c, s, pl.ds(0, 2)], pbuf_a)
    pltpu.make_async_copy(m_hbm.at[pbuf_a.at[0, 0]], rows_a, sem_a).start()
    pltpu.make_async_copy(idx_hbm.at[c, s, pl.ds(2, 2)], pbuf_b, sem_ib).start()

    def _pair(idx2):
        return idx_hbm.at[c, s, pl.ds(2 * jnp.minimum(idx2, NPAIR - 1), 2)]

    def body(i, _):
        j = 4 * i
        p = 2 * i
        pltpu.make_async_copy(m_hbm.at[pbuf_a.at[0, 0]], rows_a, sem_a).wait()
        pltpu.make_async_copy(m_hbm.at[pbuf_a.at[1, 0]], rows_b, sem_b).start()
        pltpu.sync_copy(rows_a, agg_s.at[pbuf_a.at[0, 1]], add=True)
        pltpu.make_async_copy(_pair(p + 1), pbuf_b, sem_ib).wait()
        pltpu.make_async_copy(m_hbm.at[pbuf_a.at[1, 0]], rows_b, sem_b).wait()
        pltpu.make_async_copy(m_hbm.at[pbuf_b.at[0, 0]], rows_a, sem_a).start()
        pltpu.sync_copy(rows_b, agg_s.at[pbuf_a.at[1, 1]], add=True)
        pltpu.make_async_copy(_pair(p + 2), pbuf_a, sem_ia).start()
        pltpu.make_async_copy(m_hbm.at[pbuf_b.at[0, 0]], rows_a, sem_a).wait()
        pltpu.make_async_copy(m_hbm.at[pbuf_b.at[1, 0]], rows_b, sem_b).start()
        pltpu.sync_copy(rows_a, agg_s.at[pbuf_b.at[0, 1]], add=True)
        pltpu.make_async_copy(_pair(p + 2), pbuf_a, sem_ia).wait()
        pltpu.make_async_copy(m_hbm.at[pbuf_b.at[1, 0]], rows_b, sem_b).wait()
        pltpu.make_async_copy(m_hbm.at[pbuf_a.at[0, 0]], rows_a, sem_a).start()
        pltpu.sync_copy(rows_b, agg_s.at[pbuf_b.at[1, 1]], add=True)
        pltpu.make_async_copy(_pair(p + 3), pbuf_b, sem_ib).start()
        return 0

    lax.fori_loop(0, NCHUNK // 4, body, 0)
    pltpu.make_async_copy(m_hbm.at[pbuf_a.at[0, 0]], rows_a, sem_a).wait()
    pltpu.make_async_copy(_pair(NPAIR - 1), pbuf_b, sem_ib).wait()
    plsc.subcore_barrier()
    pltpu.sync_copy(agg_s.at[pl.ds(row0, ROWS_PER_TILE)],
                    out_hbm.at[c, pl.ds(row0, ROWS_PER_TILE)])


_P = jax.lax.Precision.HIGHEST


def _dot(a, b):
    return jax.lax.dot_general(a, b, (((1,), (0,)), ((), ())),
                               precision=_P, preferred_element_type=jnp.float32)


def _tc1_body(x_ref, wm_ref, bm_ref, m1_ref):
    m1_ref[...] = jnp.maximum(_dot(x_ref[...], wm_ref[...]) + bm_ref[...], 0.0)


def _tc2_body(x_ref, a_ref, ws_ref, wa_ref, b_ref, wm_ref, bm_ref,
              h1_ref, m2_ref):
    agg = a_ref[0] + a_ref[1]
    h1 = jnp.maximum(_dot(x_ref[...], ws_ref[...]) + _dot(agg, wa_ref[...])
                     + b_ref[...], 0.0)
    h1_ref[...] = h1
    m2_ref[...] = jnp.maximum(_dot(h1, wm_ref[...]) + bm_ref[...], 0.0)


def _tc3_body(h1_ref, a_ref, n2g_ref, ws_ref, wa_ref, b_ref,
              lng_ref, lnb_ref, w1_ref, b1_ref, bng_ref, bnb_ref,
              w2_ref, b2_ref, mol_ref, out_ref):
    i = pl.program_id(0)
    agg = a_ref[0] + a_ref[1]
    h2 = jnp.maximum(_dot(h1_ref[...], ws_ref[...]) + _dot(agg, wa_ref[...])
                     + b_ref[...], 0.0)
    seg = n2g_ref[0, 0, :]
    pt = (jax.lax.broadcasted_iota(jnp.int32, (G, BN), 0)
          == seg[None, :]).astype(jnp.bfloat16)

    @pl.when(i == 0)
    def _():
        mol_ref[...] = jnp.zeros_like(mol_ref)

    mol_ref[...] += jax.lax.dot_general(
        pt, h2.astype(jnp.bfloat16), (((1,), (0,)), ((), ())),
        preferred_element_type=jnp.float32)

    @pl.when(i == GRID - 1)
    def _():
        mol = mol_ref[...]
        mu = jnp.mean(mol, axis=-1, keepdims=True)
        var = jnp.mean((mol - mu) ** 2, axis=-1, keepdims=True)
        h = (mol - mu) * jax.lax.rsqrt(var + 1e-5) * lng_ref[...] + lnb_ref[...]
        h = _dot(h, w1_ref[...]) + b1_ref[...]
        bm = jnp.mean(h, axis=0, keepdims=True)
        bv = jnp.mean((h - bm) ** 2, axis=0, keepdims=True)
        h = (h - bm) * jax.lax.rsqrt(bv + 1e-5) * bng_ref[...] + bnb_ref[...]
        h = jnp.maximum(h, 0.0)
        logit = _dot(h, w2_ref[...]) + b2_ref[...]
        out_ref[...] = jax.nn.sigmoid(logit) * 100.0


_spec_rows = pl.BlockSpec((BN, D), lambda i: (i, 0))
_spec_agg = pl.BlockSpec((NC, BN, D), lambda i: (0, i, 0))
_spec_w = pl.BlockSpec((D, D), lambda i: (0, 0))
_spec_b = pl.BlockSpec((1, D), lambda i: (0, 0))
_spec_n2g = pl.BlockSpec((1, 1, BN), lambda i: (i, 0, 0))
_spec_mol = pl.BlockSpec((G, D), lambda i: (0, 0))
_spec_out = pl.BlockSpec((G, 1), lambda i: (0, 0))


def kernel(x, edge_index, node2graph, W1_msg, b1_msg, W1_self, W1_agg, b1,
           W2_msg, b2_msg, W2_self, W2_agg, b2,
           ln_g, ln_b, fc_w1, fc_b1, bn_g, bn_b, fc_w2, fc_b2):
    f32 = jnp.float32
    src = edge_index[0]
    dst = edge_index[1]
    E_T = E // NW
    PAD_T = NCHUNK * CHUNK - E_T
    pad_src = jnp.broadcast_to(jnp.arange(PAD_T, dtype=jnp.int32), (NW, PAD_T))
    pad_dst = jnp.broadcast_to(N + jnp.arange(PAD_T, dtype=jnp.int32),
                               (NW, PAD_T))

    def _split(flat, pad):
        tiles = jnp.concatenate([flat.reshape(NW, E_T), pad], axis=1)
        return tiles.reshape(NC, NS, NCHUNK, CHUNK)

    idx_p = jnp.stack([_split(src, pad_src), _split(dst, pad_dst)], axis=3)
    n2g_p = node2graph.reshape(GRID, 1, BN)

    b1m_2 = b1_msg.reshape(1, D)
    b1_2 = b1.reshape(1, D)
    b2m_2 = b2_msg.reshape(1, D)
    b2_2 = b2.reshape(1, D)

    m1 = pl.pallas_call(
        _tc1_body,
        grid=(GRID,),
        in_specs=[_spec_rows, _spec_w, _spec_b],
        out_specs=_spec_rows,
        out_shape=jax.ShapeDtypeStruct((N, D), f32),
    )(x, W1_msg, b1m_2)

    agg1 = _sc_scatter(m1, idx_p)

    h1, m2 = pl.pallas_call(
        _tc2_body,
        grid=(GRID,),
        in_specs=[_spec_rows, _spec_agg, _spec_w, _spec_w,
                  _spec_b, _spec_w, _spec_b],
        out_specs=[_spec_rows, _spec_rows],
        out_shape=[jax.ShapeDtypeStruct((N, D), f32),
                   jax.ShapeDtypeStruct((N, D), f32)],
    )(x, agg1, W1_self, W1_agg, b1_2, W2_msg, b2m_2)

    agg2 = _sc_scatter(m2, idx_p)

    _, out = pl.pallas_call(
        _tc3_body,
        grid=(GRID,),
        in_specs=[_spec_rows, _spec_agg, _spec_n2g, _spec_w,
                  _spec_w, _spec_b,
                  _spec_b, _spec_b, _spec_w, _spec_b, _spec_b, _spec_b,
                  pl.BlockSpec((D, 1), lambda i: (0, 0)),
                  pl.BlockSpec((1, 1), lambda i: (0, 0))],
        out_specs=[_spec_mol, _spec_out],
        out_shape=[jax.ShapeDtypeStruct((G, D), f32),
                   jax.ShapeDtypeStruct((G, 1), f32)],
    )(h1, agg2, n2g_p, W2_self, W2_agg, b2_2,
      ln_g.reshape(1, D), ln_b.reshape(1, D), fc_w1, fc_b1.reshape(1, D),
      bn_g.reshape(1, D), bn_b.reshape(1, D), fc_w2, fc_b2.reshape(1, 1))

    out1 = out[:, 0]
    return (out1, out1)


# speedup vs baseline: 1.0620x; 1.0620x over previous
import functools

import jax
import jax.numpy as jnp
from jax import lax
from jax.experimental import pallas as pl
from jax.experimental.pallas import tpu as pltpu
from jax.experimental.pallas import tpu_sc as plsc

N = 10000
E = 320000
D = 128
G = 256

NC = 2
NS = 16
NW = NC * NS

CHUNK = 128
NCHUNK = 80
NPAIR = NCHUNK // 2
E_PAD = NC * NS * NCHUNK * CHUNK
N_PAD = 10240
ROWS_PER_TILE = N_PAD // NS
BN = 1280
GRID = N_PAD // BN


_sc_mesh = plsc.VectorSubcoreMesh(core_axis_name="c", subcore_axis_name="s")


@functools.partial(
    pl.kernel,
    out_type=jax.ShapeDtypeStruct((NC, N_PAD, D), jnp.float32),
    mesh=_sc_mesh,
    scratch_types=[
        pltpu.VMEM((2, 2, CHUNK), jnp.int32),
        pltpu.VMEM((2, 2, CHUNK), jnp.int32),
        pltpu.VMEM((CHUNK, D), jnp.float32),
        pltpu.VMEM((CHUNK, D), jnp.float32),
        pltpu.VMEM_SHARED((N_PAD, D), jnp.float32),
        pltpu.SemaphoreType.DMA,
        pltpu.SemaphoreType.DMA,
        pltpu.SemaphoreType.DMA,
        pltpu.SemaphoreType.DMA,
    ],
)
def _sc_scatter(m_hbm, idx_hbm, out_hbm,
                pbuf_a, pbuf_b, rows_a, rows_b, agg_s,
                sem_a, sem_b, sem_ia, sem_ib):
    c = lax.axis_index("c")
    s = lax.axis_index("s")
    row0 = s * ROWS_PER_TILE
    rows_a[...] = jnp.zeros((CHUNK, D), jnp.float32)
    for k in range(ROWS_PER_TILE // CHUNK):
        pltpu.sync_copy(rows_a, agg_s.at[pl.ds(row0 + k * CHUNK, CHUNK)])
    plsc.subcore_barrier()

    pltpu.sync_copy(idx_hbm.at[c, s, pl.ds(0, 2)], pbuf_a)
    pltpu.make_async_copy(m_hbm.at[pbuf_a.at[0, 0]], rows_a, sem_a).start()
    pltpu.make_async_copy(idx_hbm.at[c, s, pl.ds(2, 2)], pbuf_b, sem_ib).start()

    def _pair(idx2):
        return idx_hbm.at[c, s, pl.ds(2 * jnp.minimum(idx2, NPAIR - 1), 2)]

    def body(i, _):
        j = 4 * i
        p = 2 * i
        pltpu.make_async_copy(m_hbm.at[pbuf_a.at[0, 0]], rows_a, sem_a).wait()
        pltpu.make_async_copy(m_hbm.at[pbuf_a.at[1, 0]], rows_b, sem_b).start()
        pltpu.sync_copy(rows_a, agg_s.at[pbuf_a.at[0, 1]], add=True)
        pltpu.make_async_copy(_pair(p + 1), pbuf_b, sem_ib).wait()
        pltpu.make_async_copy(m_hbm.at[pbuf_a.at[1, 0]], rows_b, sem_b).wait()
        pltpu.make_async_copy(m_hbm.at[pbuf_b.at[0, 0]], rows_a, sem_a).start()
        pltpu.sync_copy(rows_b, agg_s.at[pbuf_a.at[1, 1]], add=True)
        pltpu.make_async_copy(_pair(p + 2), pbuf_a, sem_ia).start()
        pltpu.make_async_copy(m_hbm.at[pbuf_b.at[0, 0]], rows_a, sem_a).wait()
        pltpu.make_async_copy(m_hbm.at[pbuf_b.at[1, 0]], rows_b, sem_b).start()
        pltpu.sync_copy(rows_a, agg_s.at[pbuf_b.at[0, 1]], add=True)
        pltpu.make_async_copy(_pair(p + 2), pbuf_a, sem_ia).wait()
        pltpu.make_async_copy(m_hbm.at[pbuf_b.at[1, 0]], rows_b, sem_b).wait()
        pltpu.make_async_copy(m_hbm.at[pbuf_a.at[0, 0]], rows_a, sem_a).start()
        pltpu.sync_copy(rows_b, agg_s.at[pbuf_b.at[1, 1]], add=True)
        pltpu.make_async_copy(_pair(p + 3), pbuf_b, sem_ib).start()
        return 0

    lax.fori_loop(0, NCHUNK // 4, body, 0)
    pltpu.make_async_copy(m_hbm.at[pbuf_a.at[0, 0]], rows_a, sem_a).wait()
    pltpu.make_async_copy(_pair(NPAIR - 1), pbuf_b, sem_ib).wait()
    plsc.subcore_barrier()
    pltpu.sync_copy(agg_s.at[pl.ds(row0, ROWS_PER_TILE)],
                    out_hbm.at[c, pl.ds(row0, ROWS_PER_TILE)])


_P = jax.lax.Precision.HIGHEST


def _dot(a, b):
    return jax.lax.dot_general(a, b, (((1,), (0,)), ((), ())),
                               precision=_P, preferred_element_type=jnp.float32)


def _tc1_body(x_ref, wm_ref, bm_ref, m1_ref):
    m1_ref[...] = jnp.maximum(_dot(x_ref[...], wm_ref[...]) + bm_ref[...], 0.0)


def _tc2_body(x_ref, a_ref, ws_ref, wa_ref, b_ref, wm_ref, bm_ref,
              h1_ref, m2_ref):
    agg = a_ref[0] + a_ref[1]
    h1 = jnp.maximum(_dot(x_ref[...], ws_ref[...]) + _dot(agg, wa_ref[...])
                     + b_ref[...], 0.0)
    h1_ref[...] = h1
    m2_ref[...] = jnp.maximum(_dot(h1, wm_ref[...]) + bm_ref[...], 0.0)


def _tc3_body(h1_ref, a_ref, n2g_ref, ws_ref, wa_ref, b_ref,
              lng_ref, lnb_ref, w1_ref, b1_ref, bng_ref, bnb_ref,
              w2_ref, b2_ref, mol_ref, out_ref):
    i = pl.program_id(0)
    agg = a_ref[0] + a_ref[1]
    h2 = jnp.maximum(_dot(h1_ref[...], ws_ref[...]) + _dot(agg, wa_ref[...])
                     + b_ref[...], 0.0)
    seg = n2g_ref[0, 0, :]
    pt = (jax.lax.broadcasted_iota(jnp.int32, (G, BN), 0)
          == seg[None, :]).astype(jnp.bfloat16)

    @pl.when(i == 0)
    def _():
        mol_ref[...] = jnp.zeros_like(mol_ref)

    mol_ref[...] += jax.lax.dot_general(
        pt, h2.astype(jnp.bfloat16), (((1,), (0,)), ((), ())),
        preferred_element_type=jnp.float32)

    @pl.when(i == GRID - 1)
    def _():
        mol = mol_ref[...]
        mu = jnp.mean(mol, axis=-1, keepdims=True)
        var = jnp.mean((mol - mu) ** 2, axis=-1, keepdims=True)
        h = (mol - mu) * jax.lax.rsqrt(var + 1e-5) * lng_ref[...] + lnb_ref[...]
        h = _dot(h, w1_ref[...]) + b1_ref[...]
        bm = jnp.mean(h, axis=0, keepdims=True)
        bv = jnp.mean((h - bm) ** 2, axis=0, keepdims=True)
        h = (h - bm) * jax.lax.rsqrt(bv + 1e-5) * bng_ref[...] + bnb_ref[...]
        h = jnp.maximum(h, 0.0)
        logit = _dot(h, w2_ref[...]) + b2_ref[...]
        out_ref[...] = jax.nn.sigmoid(logit) * 100.0


_spec_rows = pl.BlockSpec((BN, D), lambda i: (i, 0))
_spec_agg = pl.BlockSpec((NC, BN, D), lambda i: (0, i, 0))
_spec_w = pl.BlockSpec((D, D), lambda i: (0, 0))
_spec_b = pl.BlockSpec((1, D), lambda i: (0, 0))
_spec_n2g = pl.BlockSpec((1, 1, BN), lambda i: (i, 0, 0))
_spec_mol = pl.BlockSpec((G, D), lambda i: (0, 0))
_spec_out = pl.BlockSpec((G, 1), lambda i: (0, 0))


def kernel(x, edge_index, node2graph, W1_msg, b1_msg, W1_self, W1_agg, b1,
           W2_msg, b2_msg, W2_self, W2_agg, b2,
           ln_g, ln_b, fc_w1, fc_b1, bn_g, bn_b, fc_w2, fc_b2):
    f32 = jnp.float32
    x_p = jnp.zeros((N_PAD, D), f32).at[:N].set(x)
    src = edge_index[0]
    dst = edge_index[1]
    E_T = E // NW
    PAD_T = NCHUNK * CHUNK - E_T
    pad_src = jnp.broadcast_to(jnp.arange(PAD_T, dtype=jnp.int32), (NW, PAD_T))
    pad_dst = jnp.broadcast_to(N + jnp.arange(PAD_T, dtype=jnp.int32),
                               (NW, PAD_T))

    def _split(flat, pad):
        tiles = jnp.concatenate([flat.reshape(NW, E_T), pad], axis=1)
        return tiles.reshape(NC, NS, NCHUNK, CHUNK)

    idx_p = jnp.stack([_split(src, pad_src), _split(dst, pad_dst)], axis=3)
    n2g_p = jnp.full((N_PAD,), G, jnp.int32).at[:N].set(node2graph).reshape(GRID, 1, BN)

    b1m_2 = b1_msg.reshape(1, D)
    b1_2 = b1.reshape(1, D)
    b2m_2 = b2_msg.reshape(1, D)
    b2_2 = b2.reshape(1, D)

    m1 = pl.pallas_call(
        _tc1_body,
        grid=(GRID,),
        in_specs=[_spec_rows, _spec_w, _spec_b],
        out_specs=_spec_rows,
        out_shape=jax.ShapeDtypeStruct((N_PAD, D), f32),
    )(x_p, W1_msg, b1m_2)

    agg1 = _sc_scatter(m1, idx_p)

    h1, m2 = pl.pallas_call(
        _tc2_body,
        grid=(GRID,),
        in_specs=[_spec_rows, _spec_agg, _spec_w, _spec_w,
                  _spec_b, _spec_w, _spec_b],
        out_specs=[_spec_rows, _spec_rows],
        out_shape=[jax.ShapeDtypeStruct((N_PAD, D), f32),
                   jax.ShapeDtypeStruct((N_PAD, D), f32)],
    )(x_p, agg1, W1_self, W1_agg, b1_2, W2_msg, b2m_2)

    agg2 = _sc_scatter(m2, idx_p)

    _, out = pl.pallas_call(
        _tc3_body,
        grid=(GRID,),
        in_specs=[_spec_rows, _spec_agg, _spec_n2g, _spec_w,
                  _spec_w, _spec_b,
                  _spec_b, _spec_b, _spec_w, _spec_b, _spec_b, _spec_b,
                  pl.BlockSpec((D, 1), lambda i: (0, 0)),
                  pl.BlockSpec((1, 1), lambda i: (0, 0))],
        out_specs=[_spec_mol, _spec_out],
        out_shape=[jax.ShapeDtypeStruct((G, D), f32),
                   jax.ShapeDtypeStruct((G, 1), f32)],
    )(h1, agg2, n2g_p, W2_self, W2_agg, b2_2,
      ln_g.reshape(1, D), ln_b.reshape(1, D), fc_w1, fc_b1.reshape(1, D),
      bn_g.reshape(1, D), bn_b.reshape(1, D), fc_w2, fc_b2.reshape(1, 1))

    out1 = out[:, 0]
    return (out1, out1)
